# baseline (speedup 1.0000x reference)
import dataclasses
import functools

import jax
import jax.numpy as jnp
from jax import lax
from jax.experimental import pallas as pl
from jax.experimental.pallas import tpu as pltpu
from jax.experimental.pallas import tpu_sc as plsc

N = 10000
E = 320000
D = 128
L = 16
NC = 2
NS = 16
NW = NC * NS
EPW = E // NW

CHUNK = 32
NROW = E // CHUNK
CROWS = NROW // NW
XROW = NROW - CROWS * NW
NBUF = 8
GA = 7
SD = NBUF - GA
NOUT = CROWS // NBUF
LPR = 128 // CHUNK

NVEC = EPW // L

RPS = 624
RTAIL = N - NS * RPS

_mesh = plsc.VectorSubcoreMesh(core_axis_name="c", subcore_axis_name="s")

_cp = pltpu.CompilerParams()
if "needs_layout_passes" in pltpu.CompilerParams.__dataclass_fields__:
    _cp = dataclasses.replace(_cp, needs_layout_passes=False)


@functools.partial(
    pl.kernel,
    mesh=_mesh,
    out_type=jax.ShapeDtypeStruct((NW, 2, N), jnp.float32),
    compiler_params=_cp,
    scratch_types=[
        pltpu.VMEM((EPW,), jnp.int32),
        pltpu.VMEM((EPW,), jnp.int32),
        pltpu.VMEM((N,), jnp.float32),
        pltpu.VMEM((N,), jnp.float32),
        pltpu.SemaphoreType.DMA((2,)),
    ],
)
def _sc_degrees(src_hbm, dst_hbm, out_hbm, src_v, dst_v, accs_v, accd_v, sem):
    cid = lax.axis_index("c")
    sid = lax.axis_index("s")
    wid = cid * NS + sid
    zeros = jnp.zeros((L,), jnp.float32)
    ones = jnp.ones((L,), jnp.float32)
    base = wid * EPW
    cs = pltpu.make_async_copy(src_hbm.at[pl.ds(base, EPW)], src_v, sem.at[0])
    cd = pltpu.make_async_copy(dst_hbm.at[pl.ds(base, EPW)], dst_v, sem.at[1])
    cs.start()
    cd.start()

    @pl.loop(0, N // L)
    def _(i):
        accs_v[pl.ds(i * L, L)] = zeros
        accd_v[pl.ds(i * L, L)] = zeros

    cs.wait()
    cd.wait()

    @pl.loop(0, NVEC)
    def _(j):
        sl = pl.ds(j * L, L)
        plsc.addupdate_scatter(accs_v, [src_v[sl]], ones)
        plsc.addupdate_scatter(accd_v, [dst_v[sl]], ones)

    pltpu.sync_copy(accs_v, out_hbm.at[wid, 0])
    pltpu.sync_copy(accd_v, out_hbm.at[wid, 1])


@functools.partial(
    pl.kernel,
    mesh=_mesh,
    out_type=jax.ShapeDtypeStruct((NC, N, D), jnp.float32),
    scratch_types=[
        pltpu.VMEM((CROWS // LPR, 128), jnp.int32),
    ] + [pltpu.VMEM((CHUNK,), jnp.int32)] * NBUF + [
        pltpu.VMEM((CHUNK,), jnp.int32),
        pltpu.VMEM((CHUNK,), jnp.int32),
        pltpu.VMEM((NBUF, CHUNK, D), jnp.float32),
        pltpu.VMEM_SHARED((N, D), jnp.float32),
        pltpu.SemaphoreType.DMA((NBUF,)),
        pltpu.SemaphoreType.DMA((NBUF,)),
        pltpu.SemaphoreType.DMA((NBUF,)),
        pltpu.SemaphoreType.DMA,
    ],
)
def _sc_agg128(y_hbm, src3_hbm, dst1_hbm, srcx_hbm, dstx_hbm, zero_hbm,
               out_hbm, sidx, *rest):
    didx = rest[:NBUF]
    (sx_v, dx_v, rows_v, acc_sh, gsem, isem, ssem, psem) = rest[NBUF:]
    cid = lax.axis_index("c")
    sid = lax.axis_index("s")
    wid = cid * NS + sid
    rows = pl.ds(sid * RPS, RPS)
    tail = pl.ds(NS * RPS, RTAIL)

    pre = pltpu.make_async_copy(src3_hbm.at[wid], sidx, psem)
    pre.start()
    pltpu.sync_copy(zero_hbm.at[pl.ds(0, RPS)], acc_sh.at[rows])

    @pl.when(sid == NS - 1)
    def _():
        pltpu.sync_copy(zero_hbm.at[pl.ds(0, RTAIL)], acc_sh.at[tail])

    @pl.when(wid < XROW)
    def _():
        pltpu.sync_copy(srcx_hbm.at[wid], sx_v)
        pltpu.sync_copy(dstx_hbm.at[wid], dx_v)

    pre.wait()
    plsc.subcore_barrier()

    def _gather(r, h, b):
        idx = sidx.at[r, pl.ds(h * CHUNK, CHUNK)]
        return pltpu.make_async_copy(y_hbm.at[idx], rows_v.at[b], gsem.at[b])

    def _scatter(c, b):
        del c
        return pltpu.make_async_copy(
            rows_v.at[b], acc_sh.at[didx[b]], ssem.at[b])

    def _didx_load(c, b):
        return pltpu.make_async_copy(
            dst1_hbm.at[pl.ds((wid * CROWS + c) * CHUNK, CHUNK)],
            didx[b], isem.at[b])

    for b in range(GA):
        _didx_load(b, b).start()
        _gather(b // LPR, b % LPR, b).start()

    @pl.loop(0, NOUT)
    def _(g):
        for b in range(NBUF):
            v = g * NBUF + b
            _gather(g * (NBUF // LPR) + b // LPR, b % LPR, b).wait()
            _didx_load(v, b).wait()
            _scatter(v, b).start(add=True)

            @pl.when(v >= SD)
            def _():
                _scatter(v - SD, (b + GA) % NBUF).wait()

            @pl.when(v + GA < CROWS)
            def _():
                _didx_load(v + GA, (b + GA) % NBUF).start()
                _gather(g * (NBUF // LPR) + (b + GA) // LPR, (b + GA) % LPR,
                        (b + GA) % NBUF).start()

    for k in range(SD):
        v = CROWS - SD + k
        _scatter(v, v % NBUF).wait()

    @pl.when(wid < XROW)
    def _():
        xg = pltpu.make_async_copy(y_hbm.at[sx_v], rows_v.at[0], gsem.at[0])
        xg.start()
        xg.wait()
        pltpu.sync_copy(rows_v.at[0], acc_sh.at[dx_v], add=True)

    plsc.subcore_barrier()
    pltpu.sync_copy(acc_sh.at[rows], out_hbm.at[cid, rows])

    @pl.when(sid == NS - 1)
    def _():
        pltpu.sync_copy(acc_sh.at[tail], out_hbm.at[cid, tail])


@functools.partial(
    pl.kernel,
    mesh=_mesh,
    out_type=jax.ShapeDtypeStruct((NW, N), jnp.float32),
    compiler_params=_cp,
    scratch_types=[
        pltpu.VMEM((EPW,), jnp.int32),
        pltpu.VMEM((EPW,), jnp.int32),
        pltpu.VMEM((N,), jnp.float32),
        pltpu.VMEM((N,), jnp.float32),
        pltpu.SemaphoreType.DMA((3,)),
    ],
)
def _sc_agg1(u_hbm, src_hbm, dst_hbm, out_hbm, src_v, dst_v, u_v, acc_v, sem):
    cid = lax.axis_index("c")
    sid = lax.axis_index("s")
    wid = cid * NS + sid
    zeros = jnp.zeros((L,), jnp.float32)
    base = wid * EPW
    cs = pltpu.make_async_copy(src_hbm.at[pl.ds(base, EPW)], src_v, sem.at[0])
    cd = pltpu.make_async_copy(dst_hbm.at[pl.ds(base, EPW)], dst_v, sem.at[1])
    cu = pltpu.make_async_copy(u_hbm, u_v, sem.at[2])
    cs.start()
    cd.start()
    cu.start()

    @pl.loop(0, N // L)
    def _(i):
        acc_v[pl.ds(i * L, L)] = zeros

    cs.wait()
    cd.wait()
    cu.wait()

    @pl.loop(0, NVEC)
    def _(j):
        sl = pl.ds(j * L, L)
        vals = plsc.load_gather(u_v, [src_v[sl]])
        plsc.addupdate_scatter(acc_v, [dst_v[sl]], vals)

    pltpu.sync_copy(acc_v, out_hbm.at[wid])


def _mm_body(degp_ref, x_ref, w_ref, o_ref):
    deg_out = jnp.sum(degp_ref[:, 0, :], axis=0)
    ns = lax.rsqrt(jnp.maximum(deg_out, 1.0))
    o_ref[...] = jnp.dot(x_ref[...] * ns[:, None], w_ref[...],
                         preferred_element_type=jnp.float32)


def _mid_body(p_ref, degp_ref, b1_ref, w2_ref, wl_ref, o_ref):
    deg_out = jnp.sum(degp_ref[:, 0, :], axis=0)
    deg_in = jnp.sum(degp_ref[:, 1, :], axis=0)
    nsrc = lax.rsqrt(jnp.maximum(deg_out, 1.0))
    ndst = lax.rsqrt(jnp.maximum(deg_in, 1.0))
    agg = p_ref[0] + p_ref[1]
    h1 = jnp.maximum(agg * ndst[:, None] + b1_ref[...], 0.0)
    w2l = jnp.dot(w2_ref[...], wl_ref[...],
                  preferred_element_type=jnp.float32)
    u = jnp.dot(h1, w2l, preferred_element_type=jnp.float32)[:, 0]
    o_ref[...] = u * nsrc


def _fin_body(p2_ref, degp_ref, b2_ref, wl_ref, bl_ref, o_ref):
    deg_in = jnp.sum(degp_ref[:, 1, :], axis=0)
    ndst = lax.rsqrt(jnp.maximum(deg_in, 1.0))
    agg2 = jnp.sum(p2_ref[...], axis=0)
    s = jnp.dot(b2_ref[...], wl_ref[...],
                preferred_element_type=jnp.float32)[0, 0] + bl_ref[0, 0]
    o_ref[...] = (agg2 * ndst + s)[:, None]


def kernel(x, edge_index, W1, b1, W2, b2, Wl, bl):
    src = edge_index[0].astype(jnp.int32)
    dst = edge_index[1].astype(jnp.int32)
    zero128 = jnp.zeros((RPS, D), jnp.float32)

    degp = _sc_degrees(src, dst)

    y = pl.pallas_call(
        _mm_body,
        out_shape=jax.ShapeDtypeStruct((N, D), jnp.float32),
    )(degp, x, W1)

    nmain = NW * CROWS * CHUNK
    src3 = src[:nmain].reshape(NW, CROWS // LPR, 128)
    srcx = src[nmain:].reshape(XROW, CHUNK)
    dstx = dst[nmain:].reshape(XROW, CHUNK)
    part1 = _sc_agg128(y, src3, dst, srcx, dstx, zero128)

    u = pl.pallas_call(
        _mid_body,
        out_shape=jax.ShapeDtypeStruct((N,), jnp.float32),
    )(part1, degp, b1.reshape(1, D), W2, Wl)

    part2 = _sc_agg1(u, src, dst)

    out = pl.pallas_call(
        _fin_body,
        out_shape=jax.ShapeDtypeStruct((N, 1), jnp.float32),
    )(part2, degp, b2.reshape(1, D), Wl, bl.reshape(1, 1))

    return out


# speedup vs baseline: 1.0085x; 1.0085x over previous
import dataclasses
import functools

import jax
import jax.numpy as jnp
from jax import lax
from jax.experimental import pallas as pl
from jax.experimental.pallas import tpu as pltpu
from jax.experimental.pallas import tpu_sc as plsc

N = 10000
E = 320000
D = 128
L = 16
NC = 2
NS = 16
NW = NC * NS
EPW = E // NW

CHUNK = 32
NROW = E // CHUNK
CROWS = NROW // NW
XROW = NROW - CROWS * NW
NBUF = 8
GA = 6
SD = NBUF - GA
NOUT = CROWS // NBUF
LPR = 128 // CHUNK

NVEC = EPW // L

RPS = 624
RTAIL = N - NS * RPS

_mesh = plsc.VectorSubcoreMesh(core_axis_name="c", subcore_axis_name="s")

_cp = pltpu.CompilerParams()
if "needs_layout_passes" in pltpu.CompilerParams.__dataclass_fields__:
    _cp = dataclasses.replace(_cp, needs_layout_passes=False)


@functools.partial(
    pl.kernel,
    mesh=_mesh,
    out_type=jax.ShapeDtypeStruct((NW, 2, N), jnp.float32),
    compiler_params=_cp,
    scratch_types=[
        pltpu.VMEM((EPW,), jnp.int32),
        pltpu.VMEM((EPW,), jnp.int32),
        pltpu.VMEM((N,), jnp.float32),
        pltpu.VMEM((N,), jnp.float32),
        pltpu.SemaphoreType.DMA((2,)),
    ],
)
def _sc_degrees(src_hbm, dst_hbm, out_hbm, src_v, dst_v, accs_v, accd_v, sem):
    cid = lax.axis_index("c")
    sid = lax.axis_index("s")
    wid = cid * NS + sid
    zeros = jnp.zeros((L,), jnp.float32)
    ones = jnp.ones((L,), jnp.float32)
    base = wid * EPW
    cs = pltpu.make_async_copy(src_hbm.at[pl.ds(base, EPW)], src_v, sem.at[0])
    cd = pltpu.make_async_copy(dst_hbm.at[pl.ds(base, EPW)], dst_v, sem.at[1])
    cs.start()
    cd.start()

    @pl.loop(0, N // L)
    def _(i):
        accs_v[pl.ds(i * L, L)] = zeros
        accd_v[pl.ds(i * L, L)] = zeros

    cs.wait()
    cd.wait()

    @pl.loop(0, NVEC)
    def _(j):
        sl = pl.ds(j * L, L)
        plsc.addupdate_scatter(accs_v, [src_v[sl]], ones)
        plsc.addupdate_scatter(accd_v, [dst_v[sl]], ones)

    pltpu.sync_copy(accs_v, out_hbm.at[wid, 0])
    pltpu.sync_copy(accd_v, out_hbm.at[wid, 1])


@functools.partial(
    pl.kernel,
    mesh=_mesh,
    out_type=jax.ShapeDtypeStruct((NC, N, D), jnp.float32),
    scratch_types=[
        pltpu.VMEM((CROWS // LPR, 128), jnp.int32),
    ] + [pltpu.VMEM((CHUNK,), jnp.int32)] * NBUF + [
        pltpu.VMEM((CHUNK,), jnp.int32),
        pltpu.VMEM((CHUNK,), jnp.int32),
        pltpu.VMEM((NBUF, CHUNK, D), jnp.float32),
        pltpu.VMEM_SHARED((N, D), jnp.float32),
        pltpu.SemaphoreType.DMA((NBUF,)),
        pltpu.SemaphoreType.DMA((NBUF,)),
        pltpu.SemaphoreType.DMA((NBUF,)),
        pltpu.SemaphoreType.DMA,
    ],
)
def _sc_agg128(y_hbm, src3_hbm, dst1_hbm, srcx_hbm, dstx_hbm, zero_hbm,
               out_hbm, sidx, *rest):
    didx = rest[:NBUF]
    (sx_v, dx_v, rows_v, acc_sh, gsem, isem, ssem, psem) = rest[NBUF:]
    cid = lax.axis_index("c")
    sid = lax.axis_index("s")
    wid = cid * NS + sid
    rows = pl.ds(sid * RPS, RPS)
    tail = pl.ds(NS * RPS, RTAIL)

    pre = pltpu.make_async_copy(src3_hbm.at[wid], sidx, psem)
    pre.start()
    pltpu.sync_copy(zero_hbm.at[pl.ds(0, RPS)], acc_sh.at[rows])

    @pl.when(sid == NS - 1)
    def _():
        pltpu.sync_copy(zero_hbm.at[pl.ds(0, RTAIL)], acc_sh.at[tail])

    @pl.when(wid < XROW)
    def _():
        pltpu.sync_copy(srcx_hbm.at[wid], sx_v)
        pltpu.sync_copy(dstx_hbm.at[wid], dx_v)

    pre.wait()
    plsc.subcore_barrier()

    def _gather(r, h, b):
        idx = sidx.at[r, pl.ds(h * CHUNK, CHUNK)]
        return pltpu.make_async_copy(y_hbm.at[idx], rows_v.at[b], gsem.at[b])

    def _scatter(c, b):
        del c
        return pltpu.make_async_copy(
            rows_v.at[b], acc_sh.at[didx[b]], ssem.at[b])

    def _didx_load(c, b):
        return pltpu.make_async_copy(
            dst1_hbm.at[pl.ds((wid * CROWS + c) * CHUNK, CHUNK)],
            didx[b], isem.at[b])

    for b in range(GA):
        _didx_load(b, b).start()
        _gather(b // LPR, b % LPR, b).start()

    @pl.loop(0, NOUT)
    def _(g):
        for b in range(NBUF):
            v = g * NBUF + b
            _gather(g * (NBUF // LPR) + b // LPR, b % LPR, b).wait()
            _didx_load(v, b).wait()
            _scatter(v, b).start(add=True)

            @pl.when(v >= SD)
            def _():
                _scatter(v - SD, (b + GA) % NBUF).wait()

            @pl.when(v + GA < CROWS)
            def _():
                _didx_load(v + GA, (b + GA) % NBUF).start()
                _gather(g * (NBUF // LPR) + (b + GA) // LPR, (b + GA) % LPR,
                        (b + GA) % NBUF).start()

    for k in range(SD):
        v = CROWS - SD + k
        _scatter(v, v % NBUF).wait()

    @pl.when(wid < XROW)
    def _():
        xg = pltpu.make_async_copy(y_hbm.at[sx_v], rows_v.at[0], gsem.at[0])
        xg.start()
        xg.wait()
        pltpu.sync_copy(rows_v.at[0], acc_sh.at[dx_v], add=True)

    plsc.subcore_barrier()
    pltpu.sync_copy(acc_sh.at[rows], out_hbm.at[cid, rows])

    @pl.when(sid == NS - 1)
    def _():
        pltpu.sync_copy(acc_sh.at[tail], out_hbm.at[cid, tail])


@functools.partial(
    pl.kernel,
    mesh=_mesh,
    out_type=jax.ShapeDtypeStruct((NW, N), jnp.float32),
    compiler_params=_cp,
    scratch_types=[
        pltpu.VMEM((EPW,), jnp.int32),
        pltpu.VMEM((EPW,), jnp.int32),
        pltpu.VMEM((N,), jnp.float32),
        pltpu.VMEM((N,), jnp.float32),
        pltpu.SemaphoreType.DMA((3,)),
    ],
)
def _sc_agg1(u_hbm, src_hbm, dst_hbm, out_hbm, src_v, dst_v, u_v, acc_v, sem):
    cid = lax.axis_index("c")
    sid = lax.axis_index("s")
    wid = cid * NS + sid
    zeros = jnp.zeros((L,), jnp.float32)
    base = wid * EPW
    cs = pltpu.make_async_copy(src_hbm.at[pl.ds(base, EPW)], src_v, sem.at[0])
    cd = pltpu.make_async_copy(dst_hbm.at[pl.ds(base, EPW)], dst_v, sem.at[1])
    cu = pltpu.make_async_copy(u_hbm, u_v, sem.at[2])
    cs.start()
    cd.start()
    cu.start()

    @pl.loop(0, N // L)
    def _(i):
        acc_v[pl.ds(i * L, L)] = zeros

    cs.wait()
    cd.wait()
    cu.wait()

    @pl.loop(0, NVEC)
    def _(j):
        sl = pl.ds(j * L, L)
        vals = plsc.load_gather(u_v, [src_v[sl]])
        plsc.addupdate_scatter(acc_v, [dst_v[sl]], vals)

    pltpu.sync_copy(acc_v, out_hbm.at[wid])


def _mm_body(degp_ref, x_ref, w_ref, o_ref):
    deg_out = jnp.sum(degp_ref[:, 0, :], axis=0)
    ns = lax.rsqrt(jnp.maximum(deg_out, 1.0))
    o_ref[...] = jnp.dot(x_ref[...] * ns[:, None], w_ref[...],
                         preferred_element_type=jnp.float32)


def _mid_body(p_ref, degp_ref, b1_ref, w2_ref, wl_ref, o_ref):
    deg_out = jnp.sum(degp_ref[:, 0, :], axis=0)
    deg_in = jnp.sum(degp_ref[:, 1, :], axis=0)
    nsrc = lax.rsqrt(jnp.maximum(deg_out, 1.0))
    ndst = lax.rsqrt(jnp.maximum(deg_in, 1.0))
    agg = p_ref[0] + p_ref[1]
    h1 = jnp.maximum(agg * ndst[:, None] + b1_ref[...], 0.0)
    w2l = jnp.dot(w2_ref[...], wl_ref[...],
                  preferred_element_type=jnp.float32)
    u = jnp.dot(h1, w2l, preferred_element_type=jnp.float32)[:, 0]
    o_ref[...] = u * nsrc


def _fin_body(p2_ref, degp_ref, b2_ref, wl_ref, bl_ref, o_ref):
    deg_in = jnp.sum(degp_ref[:, 1, :], axis=0)
    ndst = lax.rsqrt(jnp.maximum(deg_in, 1.0))
    agg2 = jnp.sum(p2_ref[...], axis=0)
    s = jnp.dot(b2_ref[...], wl_ref[...],
                preferred_element_type=jnp.float32)[0, 0] + bl_ref[0, 0]
    o_ref[...] = (agg2 * ndst + s)[:, None]


def kernel(x, edge_index, W1, b1, W2, b2, Wl, bl):
    src = edge_index[0].astype(jnp.int32)
    dst = edge_index[1].astype(jnp.int32)
    zero128 = jnp.zeros((RPS, D), jnp.float32)

    degp = _sc_degrees(src, dst)

    y = pl.pallas_call(
        _mm_body,
        out_shape=jax.ShapeDtypeStruct((N, D), jnp.float32),
    )(degp, x, W1)

    nmain = NW * CROWS * CHUNK
    src3 = src[:nmain].reshape(NW, CROWS // LPR, 128)
    srcx = src[nmain:].reshape(XROW, CHUNK)
    dstx = dst[nmain:].reshape(XROW, CHUNK)
    part1 = _sc_agg128(y, src3, dst, srcx, dstx, zero128)

    u = pl.pallas_call(
        _mid_body,
        out_shape=jax.ShapeDtypeStruct((N,), jnp.float32),
    )(part1, degp, b1.reshape(1, D), W2, Wl)

    part2 = _sc_agg1(u, src, dst)

    out = pl.pallas_call(
        _fin_body,
        out_shape=jax.ShapeDtypeStruct((N, 1), jnp.float32),
    )(part2, degp, b2.reshape(1, D), Wl, bl.reshape(1, 1))

    return out


# speedup vs baseline: 1.1477x; 1.1380x over previous
import dataclasses
import functools

import jax
import jax.numpy as jnp
from jax import lax
from jax.experimental import pallas as pl
from jax.experimental.pallas import tpu as pltpu
from jax.experimental.pallas import tpu_sc as plsc

N = 10000
E = 320000
D = 128
L = 16
NC = 2
NS = 16
NW = NC * NS
EPW = E // NW

CHUNK = 32
NROW = E // CHUNK
CROWS = NROW // NW
XROW = NROW - CROWS * NW
NBUF = 8
GA = 6
SD = NBUF - GA
NOUT = CROWS // NBUF
LPR = 128 // CHUNK

NVEC = EPW // L

RPS = 624
RTAIL = N - NS * RPS

_mesh = plsc.VectorSubcoreMesh(core_axis_name="c", subcore_axis_name="s")

_cp = pltpu.CompilerParams()
if "needs_layout_passes" in pltpu.CompilerParams.__dataclass_fields__:
    _cp = dataclasses.replace(_cp, needs_layout_passes=False)


@functools.partial(
    pl.kernel,
    mesh=_mesh,
    out_type=jax.ShapeDtypeStruct((NW, 2, N), jnp.float32),
    compiler_params=_cp,
    scratch_types=[
        pltpu.VMEM((EPW,), jnp.int32),
        pltpu.VMEM((EPW,), jnp.int32),
        pltpu.VMEM((N,), jnp.float32),
        pltpu.VMEM((N,), jnp.float32),
        pltpu.SemaphoreType.DMA((2,)),
    ],
)
def _sc_degrees(ei_hbm, out_hbm, src_v, dst_v, accs_v, accd_v, sem):
    cid = lax.axis_index("c")
    sid = lax.axis_index("s")
    wid = cid * NS + sid
    zeros = jnp.zeros((L,), jnp.float32)
    ones = jnp.ones((L,), jnp.float32)
    base = wid * EPW
    cs = pltpu.make_async_copy(ei_hbm.at[pl.ds(base, EPW)], src_v, sem.at[0])
    cd = pltpu.make_async_copy(ei_hbm.at[pl.ds(E + base, EPW)], dst_v, sem.at[1])
    cs.start()
    cd.start()

    @pl.loop(0, N // L)
    def _(i):
        accs_v[pl.ds(i * L, L)] = zeros
        accd_v[pl.ds(i * L, L)] = zeros

    cs.wait()
    cd.wait()

    @pl.loop(0, NVEC)
    def _(j):
        sl = pl.ds(j * L, L)
        plsc.addupdate_scatter(accs_v, [src_v[sl]], ones)
        plsc.addupdate_scatter(accd_v, [dst_v[sl]], ones)

    pltpu.sync_copy(accs_v, out_hbm.at[wid, 0])
    pltpu.sync_copy(accd_v, out_hbm.at[wid, 1])


@functools.partial(
    pl.kernel,
    mesh=_mesh,
    out_type=jax.ShapeDtypeStruct((NC, N, D), jnp.float32),
    scratch_types=[
        pltpu.VMEM((CROWS * CHUNK,), jnp.int32),
    ] + [pltpu.VMEM((CHUNK,), jnp.int32)] * NBUF + [
        pltpu.VMEM((CHUNK,), jnp.int32),
        pltpu.VMEM((CHUNK,), jnp.int32),
        pltpu.VMEM((NBUF, CHUNK, D), jnp.float32),
        pltpu.VMEM_SHARED((N, D), jnp.float32),
        pltpu.SemaphoreType.DMA((NBUF,)),
        pltpu.SemaphoreType.DMA((NBUF,)),
        pltpu.SemaphoreType.DMA((NBUF,)),
        pltpu.SemaphoreType.DMA,
    ],
)
def _sc_agg128(y_hbm, ei_hbm, zero_hbm, out_hbm, sidx, *rest):
    didx = rest[:NBUF]
    (sx_v, dx_v, rows_v, acc_sh, gsem, isem, ssem, psem) = rest[NBUF:]
    cid = lax.axis_index("c")
    sid = lax.axis_index("s")
    wid = cid * NS + sid
    rows = pl.ds(sid * RPS, RPS)
    tail = pl.ds(NS * RPS, RTAIL)

    nmain = NW * CROWS * CHUNK
    pre = pltpu.make_async_copy(
        ei_hbm.at[pl.ds(wid * CROWS * CHUNK, CROWS * CHUNK)], sidx, psem)
    pre.start()
    pltpu.sync_copy(zero_hbm.at[pl.ds(0, RPS)], acc_sh.at[rows])

    @pl.when(sid == NS - 1)
    def _():
        pltpu.sync_copy(zero_hbm.at[pl.ds(0, RTAIL)], acc_sh.at[tail])

    @pl.when(wid < XROW)
    def _():
        pltpu.sync_copy(ei_hbm.at[pl.ds(nmain + wid * CHUNK, CHUNK)], sx_v)
        pltpu.sync_copy(ei_hbm.at[pl.ds(E + nmain + wid * CHUNK, CHUNK)], dx_v)

    pre.wait()
    plsc.subcore_barrier()

    def _gather(r, h, b):
        idx = sidx.at[pl.ds((r * LPR + h) * CHUNK, CHUNK)]
        return pltpu.make_async_copy(y_hbm.at[idx], rows_v.at[b], gsem.at[b])

    def _scatter(c, b):
        del c
        return pltpu.make_async_copy(
            rows_v.at[b], acc_sh.at[didx[b]], ssem.at[b])

    def _didx_load(c, b):
        return pltpu.make_async_copy(
            ei_hbm.at[pl.ds(E + (wid * CROWS + c) * CHUNK, CHUNK)],
            didx[b], isem.at[b])

    for b in range(GA):
        _didx_load(b, b).start()
        _gather(b // LPR, b % LPR, b).start()

    @pl.loop(0, NOUT)
    def _(g):
        for b in range(NBUF):
            v = g * NBUF + b
            _gather(g * (NBUF // LPR) + b // LPR, b % LPR, b).wait()
            _didx_load(v, b).wait()
            _scatter(v, b).start(add=True)

            @pl.when(v >= SD)
            def _():
                _scatter(v - SD, (b + GA) % NBUF).wait()

            @pl.when(v + GA < CROWS)
            def _():
                _didx_load(v + GA, (b + GA) % NBUF).start()
                _gather(g * (NBUF // LPR) + (b + GA) // LPR, (b + GA) % LPR,
                        (b + GA) % NBUF).start()

    for k in range(SD):
        v = CROWS - SD + k
        _scatter(v, v % NBUF).wait()

    @pl.when(wid < XROW)
    def _():
        xg = pltpu.make_async_copy(y_hbm.at[sx_v], rows_v.at[0], gsem.at[0])
        xg.start()
        xg.wait()
        pltpu.sync_copy(rows_v.at[0], acc_sh.at[dx_v], add=True)

    plsc.subcore_barrier()
    pltpu.sync_copy(acc_sh.at[rows], out_hbm.at[cid, rows])

    @pl.when(sid == NS - 1)
    def _():
        pltpu.sync_copy(acc_sh.at[tail], out_hbm.at[cid, tail])


@functools.partial(
    pl.kernel,
    mesh=_mesh,
    out_type=jax.ShapeDtypeStruct((NW, N), jnp.float32),
    compiler_params=_cp,
    scratch_types=[
        pltpu.VMEM((EPW,), jnp.int32),
        pltpu.VMEM((EPW,), jnp.int32),
        pltpu.VMEM((N,), jnp.float32),
        pltpu.VMEM((N,), jnp.float32),
        pltpu.SemaphoreType.DMA((3,)),
    ],
)
def _sc_agg1(u_hbm, ei_hbm, out_hbm, src_v, dst_v, u_v, acc_v, sem):
    cid = lax.axis_index("c")
    sid = lax.axis_index("s")
    wid = cid * NS + sid
    zeros = jnp.zeros((L,), jnp.float32)
    base = wid * EPW
    cs = pltpu.make_async_copy(ei_hbm.at[pl.ds(base, EPW)], src_v, sem.at[0])
    cd = pltpu.make_async_copy(ei_hbm.at[pl.ds(E + base, EPW)], dst_v, sem.at[1])
    cu = pltpu.make_async_copy(u_hbm, u_v, sem.at[2])
    cs.start()
    cd.start()
    cu.start()

    @pl.loop(0, N // L)
    def _(i):
        acc_v[pl.ds(i * L, L)] = zeros

    cs.wait()
    cd.wait()
    cu.wait()

    @pl.loop(0, NVEC)
    def _(j):
        sl = pl.ds(j * L, L)
        vals = plsc.load_gather(u_v, [src_v[sl]])
        plsc.addupdate_scatter(acc_v, [dst_v[sl]], vals)

    pltpu.sync_copy(acc_v, out_hbm.at[wid])


def _mm_body(degp_ref, x_ref, w_ref, o_ref):
    deg_out = jnp.sum(degp_ref[:, 0, :], axis=0)
    ns = lax.rsqrt(jnp.maximum(deg_out, 1.0))
    o_ref[...] = jnp.dot(x_ref[...] * ns[:, None], w_ref[...],
                         preferred_element_type=jnp.float32)


def _mid_body(p_ref, degp_ref, b1_ref, w2_ref, wl_ref, o_ref):
    deg_out = jnp.sum(degp_ref[:, 0, :], axis=0)
    deg_in = jnp.sum(degp_ref[:, 1, :], axis=0)
    nsrc = lax.rsqrt(jnp.maximum(deg_out, 1.0))
    ndst = lax.rsqrt(jnp.maximum(deg_in, 1.0))
    agg = p_ref[0] + p_ref[1]
    h1 = jnp.maximum(agg * ndst[:, None] + b1_ref[...], 0.0)
    w2l = jnp.dot(w2_ref[...], wl_ref[...],
                  preferred_element_type=jnp.float32)
    u = jnp.dot(h1, w2l, preferred_element_type=jnp.float32)[:, 0]
    o_ref[...] = u * nsrc


def _fin_body(p2_ref, degp_ref, b2_ref, wl_ref, bl_ref, o_ref):
    deg_in = jnp.sum(degp_ref[:, 1, :], axis=0)
    ndst = lax.rsqrt(jnp.maximum(deg_in, 1.0))
    agg2 = jnp.sum(p2_ref[...], axis=0)
    s = jnp.dot(b2_ref[...], wl_ref[...],
                preferred_element_type=jnp.float32)[0, 0] + bl_ref[0, 0]
    o_ref[...] = (agg2 * ndst + s)[:, None]


def kernel(x, edge_index, W1, b1, W2, b2, Wl, bl):
    ei = edge_index.astype(jnp.int32).reshape(2 * E)
    zero128 = jnp.zeros((RPS, D), jnp.float32)

    degp = _sc_degrees(ei)

    y = pl.pallas_call(
        _mm_body,
        out_shape=jax.ShapeDtypeStruct((N, D), jnp.float32),
    )(degp, x, W1)

    part1 = _sc_agg128(y, ei, zero128)

    u = pl.pallas_call(
        _mid_body,
        out_shape=jax.ShapeDtypeStruct((N,), jnp.float32),
    )(part1, degp, b1.reshape(1, D), W2, Wl)

    part2 = _sc_agg1(u, ei)

    out = pl.pallas_call(
        _fin_body,
        out_shape=jax.ShapeDtypeStruct((N, 1), jnp.float32),
    )(part2, degp, b2.reshape(1, D), Wl, bl.reshape(1, 1))

    return out
